# Initial kernel scaffold; baseline (speedup 1.0000x reference)
import jax
import jax.numpy as jnp
from jax.experimental import pallas as pl


def kernel(profile, context, item_indices, item_vectors, profile_table, context_table, W1, b1, W2, b2):
    raise NotImplementedError("write your pallas kernel here")


# speedup vs baseline: 68.6517x; 68.6517x over previous
import functools

import jax
import jax.numpy as jnp
from jax import lax
from jax.experimental import pallas as pl
from jax.experimental.pallas import tpu as pltpu
from jax.experimental.pallas import tpu_sc as plsc

B = 1024
LH = 50
NP = 5
NCX = 4
D = 64
NI = 100000
TOPK = 100

NW = 32
RPW = B // NW
LANES = 16

VB = 2048
NPAD = 102400
NBLK = NPAD // VB
TPB = VB // 128
NCM = NPAD // LANES
NL2 = NCM // LANES
NL3 = 32
NEG = -1e30
BIGI = 1 << 30


def _iota16():
    return lax.broadcasted_iota(jnp.int32, (LANES,), 0)


def _gather_body(item_idx_hbm, prof_hbm, ctx_hbm, ivec_hbm, ptab_hbm, ctab_hbm,
                 items_out, prof_out, ctx_out, idx_v, pidx_v, cidx_v,
                 rows_v, prow_v, crow_v, sem):
    wid = lax.axis_index("s") * 2 + lax.axis_index("c")
    base = wid * RPW
    pltpu.sync_copy(item_idx_hbm.at[pl.ds(base, RPW)], idx_v)
    pltpu.sync_copy(prof_hbm.at[pl.ds(base, RPW)], pidx_v)
    pltpu.sync_copy(ctx_hbm.at[pl.ds(base, RPW)], cidx_v)

    def row(r, _):
        b = base + r
        pltpu.async_copy(ivec_hbm.at[idx_v.at[r]], rows_v, sem).wait()
        pltpu.sync_copy(rows_v, items_out.at[b])
        pltpu.async_copy(ptab_hbm.at[pidx_v.at[r]], prow_v, sem).wait()
        pltpu.sync_copy(prow_v, prof_out.at[b])
        pltpu.async_copy(ctab_hbm.at[cidx_v.at[r]], crow_v, sem).wait()
        pltpu.sync_copy(crow_v, ctx_out.at[b])
        return _

    lax.fori_loop(0, RPW, row, None)


def _sc_gather(item_indices, profile, context, item_vectors, profile_table, context_table):
    mesh = plsc.VectorSubcoreMesh(core_axis_name="c", subcore_axis_name="s")
    f = pl.kernel(
        _gather_body,
        out_type=[
            jax.ShapeDtypeStruct((B, LH, 128), jnp.float32),
            jax.ShapeDtypeStruct((B, NP, 128), jnp.float32),
            jax.ShapeDtypeStruct((B, NCX, 128), jnp.float32),
        ],
        mesh=mesh,
        compiler_params=pltpu.CompilerParams(needs_layout_passes=False),
        scratch_types=[
            pltpu.VMEM((RPW, LH), jnp.int32),
            pltpu.VMEM((RPW, NP), jnp.int32),
            pltpu.VMEM((RPW, NCX), jnp.int32),
            pltpu.VMEM((LH, 128), jnp.float32),
            pltpu.VMEM((NP, 128), jnp.float32),
            pltpu.VMEM((NCX, 128), jnp.float32),
            pltpu.SemaphoreType.DMA,
        ],
    )
    return f(item_indices, profile, context, item_vectors, profile_table, context_table)


BBLK = 128

def _pool_body(items_ref, prof_ref, ctx_ref, w1_ref, b1_ref, w2_ref, b2_ref, u_ref):
    p = jnp.mean(prof_ref[...][:, :, 0:D], axis=1)
    c = jnp.mean(ctx_ref[...][:, :, 0:D], axis=1)
    it = jnp.mean(items_ref[...][:, :, 0:D], axis=1)
    h = jnp.concatenate([p, c, it], axis=-1)
    h = jnp.maximum(jnp.dot(h, w1_ref[...],
                            preferred_element_type=jnp.float32) + b1_ref[...], 0.0)
    u_ref[...] = jnp.dot(h, w2_ref[...],
                         preferred_element_type=jnp.float32) + b2_ref[...]


def _tc_pool(items_g, prof_g, ctx_g, W1, b1, W2, b2):
    return pl.pallas_call(
        _pool_body,
        grid=(B // BBLK,),
        in_specs=[
            pl.BlockSpec((BBLK, LH, 128), lambda i: (i, 0, 0)),
            pl.BlockSpec((BBLK, NP, 128), lambda i: (i, 0, 0)),
            pl.BlockSpec((BBLK, NCX, 128), lambda i: (i, 0, 0)),
            pl.BlockSpec((3 * D, 2 * D), lambda i: (0, 0)),
            pl.BlockSpec((1, 2 * D), lambda i: (0, 0)),
            pl.BlockSpec((2 * D, D), lambda i: (0, 0)),
            pl.BlockSpec((1, D), lambda i: (0, 0)),
        ],
        out_specs=pl.BlockSpec((BBLK, D), lambda i: (i, 0)),
        out_shape=jax.ShapeDtypeStruct((B, D), jnp.float32),
    )(items_g, prof_g, ctx_g, W1, b1, W2, b2)


def _score_body(u_in_ref, vt_ref, scores_ref, cm_ref):
    j = pl.program_id(0)
    s = jnp.dot(u_in_ref[...], vt_ref[...], preferred_element_type=jnp.float32)
    col = j * VB + lax.broadcasted_iota(jnp.int32, (B, VB), 1)
    s = jnp.where(col < NI, s, NEG)
    scores_ref[...] = s
    cm = s[:, 0:128]
    for t in range(1, TPB):
        cm = jnp.maximum(cm, s[:, t * 128:(t + 1) * 128])
    cm_ref[...] = cm


def _tc_score(u, vt):
    return pl.pallas_call(
        _score_body,
        grid=(NBLK,),
        in_specs=[
            pl.BlockSpec((B, D), lambda j: (0, 0)),
            pl.BlockSpec((D, VB), lambda j: (0, j)),
        ],
        out_specs=[
            pl.BlockSpec((B, VB), lambda j: (0, j)),
            pl.BlockSpec((B, 128), lambda j: (0, j)),
        ],
        out_shape=[
            jax.ShapeDtypeStruct((B, NPAD), jnp.float32),
            jax.ShapeDtypeStruct((B, NCM), jnp.float32),
        ],
    )(u, vt)


def _select_body(scores_hbm, cm_hbm, rec_hbm, srow, cm_v, l2_v, l3_v, out_v, sem):
    wid = lax.axis_index("s") * 2 + lax.axis_index("c")
    iota = _iota16()

    def row(r, _):
        b = wid * RPW + r
        pltpu.sync_copy(scores_hbm.at[b], srow)
        pltpu.sync_copy(cm_hbm.at[b], cm_v)
        def bl2(g, _c):
            v = cm_v[pl.ds(g * LANES, LANES)]
            m = jnp.max(v, axis=0)
            ov = l2_v[pl.ds((g // LANES) * LANES, LANES)]
            l2_v[pl.ds((g // LANES) * LANES, LANES)] = jnp.where(
                iota == g % LANES, m, ov)
            return _c
        lax.fori_loop(0, NL2, bl2, None)

        def bl3(g, _c):
            v = l2_v[pl.ds(g * LANES, LANES)]
            m = jnp.max(v, axis=0)
            ov = l3_v[pl.ds((g // LANES) * LANES, LANES)]
            l3_v[pl.ds((g // LANES) * LANES, LANES)] = jnp.where(
                iota == g % LANES, m, ov)
            return _c
        l3_v[pl.ds(0, LANES)] = jnp.full((LANES,), NEG, jnp.float32)
        l3_v[pl.ds(LANES, LANES)] = jnp.full((LANES,), NEG, jnp.float32)
        lax.fori_loop(0, NL2 // LANES, bl3, None)

        def extract(k, _c):
            v0 = l3_v[pl.ds(0, LANES)]
            v1 = l3_v[pl.ds(LANES, LANES)]
            m = jnp.max(jnp.maximum(v0, v1), axis=0)
            c0 = jnp.where(v0 == m, iota, BIGI)
            c1 = jnp.where(v1 == m, iota + LANES, BIGI)
            g3 = jnp.min(jnp.minimum(c0, c1), axis=0)
            v = l2_v[pl.ds(g3 * LANES, LANES)]
            g2 = g3 * LANES + jnp.min(jnp.where(v == m, iota, BIGI), axis=0)
            cv = cm_v[pl.ds(g2 * LANES, LANES)]
            cl = jnp.min(jnp.where(cv == m, iota, BIGI), axis=0)
            cc = g2 * LANES + cl
            jb = cc // 128
            g = cc % 128
            eidx = jb * VB + g + iota * 128
            ev = plsc.load_gather(srow, [eidx])
            t = jnp.min(jnp.where(ev == m, iota, BIGI), axis=0)
            e = jb * VB + t * 128 + g
            kb = (k // LANES) * LANES
            ovv = out_v[pl.ds(kb, LANES)]
            out_v[pl.ds(kb, LANES)] = jnp.where(iota == k - kb, e, ovv)
            ev2 = jnp.where(iota == t, NEG, ev)
            plsc.store_scatter(srow, [eidx], ev2)
            m1 = jnp.max(ev2, axis=0)
            cv2 = jnp.where(iota == cl, m1, cv)
            cm_v[pl.ds(g2 * LANES, LANES)] = cv2
            m2 = jnp.max(cv2, axis=0)
            l2v2 = jnp.where(iota == g2 - g3 * LANES, m2, v)
            l2_v[pl.ds(g3 * LANES, LANES)] = l2v2
            m3 = jnp.max(l2v2, axis=0)
            l3b = (g3 // LANES) * LANES
            l3o = l3_v[pl.ds(l3b, LANES)]
            l3_v[pl.ds(l3b, LANES)] = jnp.where(iota == g3 - l3b, m3, l3o)
            return _c

        lax.fori_loop(0, TOPK, extract, None)
        pltpu.sync_copy(out_v, rec_hbm.at[b])
        return _

    lax.fori_loop(0, RPW, row, None)


def _sc_select(scores, cm):
    mesh = plsc.VectorSubcoreMesh(core_axis_name="c", subcore_axis_name="s")
    f = pl.kernel(
        _select_body,
        out_type=jax.ShapeDtypeStruct((B, 128), jnp.int32),
        mesh=mesh,
        compiler_params=pltpu.CompilerParams(needs_layout_passes=False),
        scratch_types=[
            pltpu.VMEM((NPAD,), jnp.float32),
            pltpu.VMEM((NCM,), jnp.float32),
            pltpu.VMEM((NL2,), jnp.float32),
            pltpu.VMEM((NL3,), jnp.float32),
            pltpu.VMEM((128,), jnp.int32),
            pltpu.SemaphoreType.DMA,
        ],
    )
    return f(scores, cm)


def kernel(profile, context, item_indices, item_vectors, profile_table,
           context_table, W1, b1, W2, b2):
    pad128 = lambda t: jnp.pad(t, ((0, 0), (0, 128 - D)))
    items_g, prof_g, ctx_g = _sc_gather(
        item_indices, profile, context, pad128(item_vectors),
        pad128(profile_table), pad128(context_table))
    u = _tc_pool(items_g, prof_g, ctx_g, W1, b1.reshape(1, -1),
                 W2, b2.reshape(1, -1))
    vt = jnp.pad(item_vectors, ((0, NPAD - NI), (0, 0))).T
    scores, cm = _tc_score(u, vt)
    rec = _sc_select(scores, cm)
    return rec[:, :TOPK]
